# baseline (speedup 1.0000x reference)
import jax
import jax.numpy as jnp
from jax.experimental import pallas as pl
from jax.experimental.pallas import tpu as pltpu

_N = 5000
_BB = 256
_NBLK = 20
_NPAD = _BB * _NBLK
_IOU_THRESH = 0.5


def _iou_block(rows, cols):
    rx1 = rows[:, 0:1]
    ry1 = rows[:, 1:2]
    rx2 = rows[:, 2:3]
    ry2 = rows[:, 3:4]
    cx1 = cols[0:1, :]
    cy1 = cols[1:2, :]
    cx2 = cols[2:3, :]
    cy2 = cols[3:4, :]
    r_area = (rx2 - rx1) * (ry2 - ry1)
    c_area = (cx2 - cx1) * (cy2 - cy1)
    w = jnp.maximum(jnp.minimum(rx2, cx2) - jnp.maximum(rx1, cx1), 0.0)
    h = jnp.maximum(jnp.minimum(ry2, cy2) - jnp.maximum(ry1, cy1), 0.0)
    inter = w * h
    union = (r_area + c_area) - inter
    return inter / union


def _nms_kernel(rows_ref, cols_ref, out_ref, accr_ref, accc_ref):
    accr_ref[...] = jnp.zeros_like(accr_ref)
    accc_ref[...] = jnp.zeros_like(accc_ref)

    def offdiag(bi, bj):
        rows = rows_ref[bi]
        cols = cols_ref[bj]
        rs = rows[:, 4:5]
        cs = cols[4:5, :]
        hot = _iou_block(rows, cols) > _IOU_THRESH
        col_beats_row = cs >= rs
        row_sup = jnp.any(hot & col_beats_row, axis=1, keepdims=True)
        col_sup = jnp.any(hot & jnp.logical_not(col_beats_row), axis=0,
                          keepdims=True)
        accr_ref[bi] = jnp.maximum(accr_ref[bi],
                                   jnp.where(row_sup, 1.0, 0.0))
        accc_ref[bj] = jnp.maximum(accc_ref[bj],
                                   jnp.where(col_sup, 1.0, 0.0))
        return 0

    def diag(bi):
        rows = rows_ref[bi]
        cols = cols_ref[bi]
        rs = rows[:, 4:5]
        cs = cols[4:5, :]
        hot = _iou_block(rows, cols) > _IOU_THRESH
        li = jax.lax.broadcasted_iota(jnp.int32, (_BB, 1), 0)
        lj = jax.lax.broadcasted_iota(jnp.int32, (1, _BB), 1)
        col_beats_row = (cs > rs) | ((cs == rs) & (lj < li))
        row_sup = jnp.any(hot & col_beats_row, axis=1, keepdims=True)
        accr_ref[bi] = jnp.maximum(accr_ref[bi],
                                   jnp.where(row_sup, 1.0, 0.0))
        return 0

    def outer(bi, carry):
        jax.lax.fori_loop(0, bi, lambda bj, c: offdiag(bi, bj), 0)
        return diag(bi)

    jax.lax.fori_loop(0, _NBLK, outer, 0)

    li = jax.lax.broadcasted_iota(jnp.int32, (_BB, _BB), 0)
    lj = jax.lax.broadcasted_iota(jnp.int32, (_BB, _BB), 1)
    eye = (li == lj).astype(jnp.float32)

    def writeout(b, carry):
        cc = jnp.sum(eye * accc_ref[b], axis=1, keepdims=True)
        sup = jnp.maximum(accr_ref[b], cc)
        out_ref[b] = jnp.where(sup > 0.0, 0.0, rows_ref[b])
        return 0

    jax.lax.fori_loop(0, _NBLK, writeout, 0)


def kernel(boxes, scores):
    data = jnp.zeros((_NPAD, 8), dtype=jnp.float32)
    data = data.at[:_N, 0:4].set(boxes)
    data = data.at[:_N, 4].set(scores)
    rows = data.reshape(_NBLK, _BB, 8)
    cols = jnp.moveaxis(data.T.reshape(8, _NBLK, _BB), 1, 0)

    out = pl.pallas_call(
        _nms_kernel,
        out_shape=jax.ShapeDtypeStruct((_NBLK, _BB, 8), jnp.float32),
        scratch_shapes=[
            pltpu.VMEM((_NBLK, _BB, 1), jnp.float32),
            pltpu.VMEM((_NBLK, 1, _BB), jnp.float32),
        ],
    )(rows, cols)

    return out.reshape(_NPAD, 8)[:_N, :5]


# speedup vs baseline: 1.5336x; 1.5336x over previous
import jax
import jax.numpy as jnp
from jax.experimental import pallas as pl
from jax.experimental.pallas import tpu as pltpu

_N = 5000
_BB = 512
_NBLK = 10
_NPAD = _BB * _NBLK
_IOU_THRESH = 0.5


def _iou_block(rows, cols):
    rx1 = rows[:, 0:1]
    ry1 = rows[:, 1:2]
    rx2 = rows[:, 2:3]
    ry2 = rows[:, 3:4]
    cx1 = cols[0:1, :]
    cy1 = cols[1:2, :]
    cx2 = cols[2:3, :]
    cy2 = cols[3:4, :]
    r_area = (rx2 - rx1) * (ry2 - ry1)
    c_area = (cx2 - cx1) * (cy2 - cy1)
    w = jnp.maximum(jnp.minimum(rx2, cx2) - jnp.maximum(rx1, cx1), 0.0)
    h = jnp.maximum(jnp.minimum(ry2, cy2) - jnp.maximum(ry1, cy1), 0.0)
    inter = w * h
    union = (r_area + c_area) - inter
    return inter / union


def _nms_kernel(rows_ref, cols_ref, out_ref, accr_ref, accc_ref):
    accr_ref[...] = jnp.zeros_like(accr_ref)
    accc_ref[...] = jnp.zeros_like(accc_ref)

    def offdiag(bi, bj):
        rows = rows_ref[bi]
        cols = cols_ref[bj]
        rs = rows[:, 4:5]
        cs = cols[4:5, :]
        hot = _iou_block(rows, cols) > _IOU_THRESH
        col_beats_row = cs >= rs
        row_sup = jnp.any(hot & col_beats_row, axis=1, keepdims=True)
        col_sup = jnp.any(hot & jnp.logical_not(col_beats_row), axis=0,
                          keepdims=True)
        accr_ref[bi] = jnp.maximum(accr_ref[bi],
                                   jnp.where(row_sup, 1.0, 0.0))
        accc_ref[bj] = jnp.maximum(accc_ref[bj],
                                   jnp.where(col_sup, 1.0, 0.0))
        return 0

    def diag(bi):
        rows = rows_ref[bi]
        cols = cols_ref[bi]
        rs = rows[:, 4:5]
        cs = cols[4:5, :]
        hot = _iou_block(rows, cols) > _IOU_THRESH
        li = jax.lax.broadcasted_iota(jnp.int32, (_BB, 1), 0)
        lj = jax.lax.broadcasted_iota(jnp.int32, (1, _BB), 1)
        col_beats_row = (cs > rs) | ((cs == rs) & (lj < li))
        row_sup = jnp.any(hot & col_beats_row, axis=1, keepdims=True)
        accr_ref[bi] = jnp.maximum(accr_ref[bi],
                                   jnp.where(row_sup, 1.0, 0.0))
        return 0

    def outer(bi, carry):
        jax.lax.fori_loop(0, bi, lambda bj, c: offdiag(bi, bj), 0)
        return diag(bi)

    jax.lax.fori_loop(0, _NBLK, outer, 0)

    li = jax.lax.broadcasted_iota(jnp.int32, (_BB, _BB), 0)
    lj = jax.lax.broadcasted_iota(jnp.int32, (_BB, _BB), 1)
    eye = (li == lj).astype(jnp.float32)

    def writeout(b, carry):
        cc = jnp.sum(eye * accc_ref[b], axis=1, keepdims=True)
        sup = jnp.maximum(accr_ref[b], cc)
        out_ref[b] = jnp.where(sup > 0.0, 0.0, rows_ref[b])
        return 0

    jax.lax.fori_loop(0, _NBLK, writeout, 0)


def kernel(boxes, scores):
    data = jnp.zeros((_NPAD, 8), dtype=jnp.float32)
    data = data.at[:_N, 0:4].set(boxes)
    data = data.at[:_N, 4].set(scores)
    rows = data.reshape(_NBLK, _BB, 8)
    cols = jnp.moveaxis(data.T.reshape(8, _NBLK, _BB), 1, 0)

    out = pl.pallas_call(
        _nms_kernel,
        out_shape=jax.ShapeDtypeStruct((_NBLK, _BB, 8), jnp.float32),
        scratch_shapes=[
            pltpu.VMEM((_NBLK, _BB, 1), jnp.float32),
            pltpu.VMEM((_NBLK, 1, _BB), jnp.float32),
        ],
    )(rows, cols)

    return out.reshape(_NPAD, 8)[:_N, :5]


# speedup vs baseline: 1.7295x; 1.1277x over previous
import jax
import jax.numpy as jnp
from jax.experimental import pallas as pl
from jax.experimental.pallas import tpu as pltpu

_N = 5000
_BB = 1024
_NBLK = 5
_NPAD = _BB * _NBLK
_IOU_THRESH = 0.5


def _iou_block(rows, cols):
    rx1 = rows[:, 0:1]
    ry1 = rows[:, 1:2]
    rx2 = rows[:, 2:3]
    ry2 = rows[:, 3:4]
    cx1 = cols[0:1, :]
    cy1 = cols[1:2, :]
    cx2 = cols[2:3, :]
    cy2 = cols[3:4, :]
    r_area = (rx2 - rx1) * (ry2 - ry1)
    c_area = (cx2 - cx1) * (cy2 - cy1)
    w = jnp.maximum(jnp.minimum(rx2, cx2) - jnp.maximum(rx1, cx1), 0.0)
    h = jnp.maximum(jnp.minimum(ry2, cy2) - jnp.maximum(ry1, cy1), 0.0)
    inter = w * h
    union = (r_area + c_area) - inter
    return inter / union


def _nms_kernel(rows_ref, cols_ref, out_ref, accr_ref, accc_ref):
    accr_ref[...] = jnp.zeros_like(accr_ref)
    accc_ref[...] = jnp.zeros_like(accc_ref)

    def offdiag(bi, bj):
        rows = rows_ref[bi]
        cols = cols_ref[bj]
        rs = rows[:, 4:5]
        cs = cols[4:5, :]
        hot = _iou_block(rows, cols) > _IOU_THRESH
        col_beats_row = cs >= rs
        row_sup = jnp.any(hot & col_beats_row, axis=1, keepdims=True)
        col_sup = jnp.any(hot & jnp.logical_not(col_beats_row), axis=0,
                          keepdims=True)
        accr_ref[bi] = jnp.maximum(accr_ref[bi],
                                   jnp.where(row_sup, 1.0, 0.0))
        accc_ref[bj] = jnp.maximum(accc_ref[bj],
                                   jnp.where(col_sup, 1.0, 0.0))
        return 0

    def diag(bi):
        rows = rows_ref[bi]
        cols = cols_ref[bi]
        rs = rows[:, 4:5]
        cs = cols[4:5, :]
        hot = _iou_block(rows, cols) > _IOU_THRESH
        li = jax.lax.broadcasted_iota(jnp.int32, (_BB, 1), 0)
        lj = jax.lax.broadcasted_iota(jnp.int32, (1, _BB), 1)
        col_beats_row = (cs > rs) | ((cs == rs) & (lj < li))
        row_sup = jnp.any(hot & col_beats_row, axis=1, keepdims=True)
        accr_ref[bi] = jnp.maximum(accr_ref[bi],
                                   jnp.where(row_sup, 1.0, 0.0))
        return 0

    def outer(bi, carry):
        jax.lax.fori_loop(0, bi, lambda bj, c: offdiag(bi, bj), 0)
        return diag(bi)

    jax.lax.fori_loop(0, _NBLK, outer, 0)

    li = jax.lax.broadcasted_iota(jnp.int32, (_BB, _BB), 0)
    lj = jax.lax.broadcasted_iota(jnp.int32, (_BB, _BB), 1)
    eye = (li == lj).astype(jnp.float32)

    def writeout(b, carry):
        cc = jnp.sum(eye * accc_ref[b], axis=1, keepdims=True)
        sup = jnp.maximum(accr_ref[b], cc)
        out_ref[b] = jnp.where(sup > 0.0, 0.0, rows_ref[b])
        return 0

    jax.lax.fori_loop(0, _NBLK, writeout, 0)


def kernel(boxes, scores):
    data = jnp.zeros((_NPAD, 8), dtype=jnp.float32)
    data = data.at[:_N, 0:4].set(boxes)
    data = data.at[:_N, 4].set(scores)
    rows = data.reshape(_NBLK, _BB, 8)
    cols = jnp.moveaxis(data.T.reshape(8, _NBLK, _BB), 1, 0)

    out = pl.pallas_call(
        _nms_kernel,
        out_shape=jax.ShapeDtypeStruct((_NBLK, _BB, 8), jnp.float32),
        scratch_shapes=[
            pltpu.VMEM((_NBLK, _BB, 1), jnp.float32),
            pltpu.VMEM((_NBLK, 1, _BB), jnp.float32),
        ],
    )(rows, cols)

    return out.reshape(_NPAD, 8)[:_N, :5]


# speedup vs baseline: 1.7307x; 1.0007x over previous
import jax
import jax.numpy as jnp
from jax.experimental import pallas as pl
from jax.experimental.pallas import tpu as pltpu

_N = 5000
_BB = 1280
_NBLK = 4
_NPAD = _BB * _NBLK
_IOU_THRESH = 0.5


def _iou_block(rows, cols):
    rx1 = rows[:, 0:1]
    ry1 = rows[:, 1:2]
    rx2 = rows[:, 2:3]
    ry2 = rows[:, 3:4]
    cx1 = cols[0:1, :]
    cy1 = cols[1:2, :]
    cx2 = cols[2:3, :]
    cy2 = cols[3:4, :]
    r_area = (rx2 - rx1) * (ry2 - ry1)
    c_area = (cx2 - cx1) * (cy2 - cy1)
    w = jnp.maximum(jnp.minimum(rx2, cx2) - jnp.maximum(rx1, cx1), 0.0)
    h = jnp.maximum(jnp.minimum(ry2, cy2) - jnp.maximum(ry1, cy1), 0.0)
    inter = w * h
    union = (r_area + c_area) - inter
    return inter / union


def _nms_kernel(rows_ref, cols_ref, out_ref, accr_ref, accc_ref):
    accr_ref[...] = jnp.zeros_like(accr_ref)
    accc_ref[...] = jnp.zeros_like(accc_ref)

    def offdiag(bi, bj):
        rows = rows_ref[bi]
        cols = cols_ref[bj]
        rs = rows[:, 4:5]
        cs = cols[4:5, :]
        hot = _iou_block(rows, cols) > _IOU_THRESH
        col_beats_row = cs >= rs
        row_sup = jnp.any(hot & col_beats_row, axis=1, keepdims=True)
        col_sup = jnp.any(hot & jnp.logical_not(col_beats_row), axis=0,
                          keepdims=True)
        accr_ref[bi] = jnp.maximum(accr_ref[bi],
                                   jnp.where(row_sup, 1.0, 0.0))
        accc_ref[bj] = jnp.maximum(accc_ref[bj],
                                   jnp.where(col_sup, 1.0, 0.0))
        return 0

    def diag(bi):
        rows = rows_ref[bi]
        cols = cols_ref[bi]
        rs = rows[:, 4:5]
        cs = cols[4:5, :]
        hot = _iou_block(rows, cols) > _IOU_THRESH
        li = jax.lax.broadcasted_iota(jnp.int32, (_BB, 1), 0)
        lj = jax.lax.broadcasted_iota(jnp.int32, (1, _BB), 1)
        col_beats_row = (cs > rs) | ((cs == rs) & (lj < li))
        row_sup = jnp.any(hot & col_beats_row, axis=1, keepdims=True)
        accr_ref[bi] = jnp.maximum(accr_ref[bi],
                                   jnp.where(row_sup, 1.0, 0.0))
        return 0

    def outer(bi, carry):
        jax.lax.fori_loop(0, bi, lambda bj, c: offdiag(bi, bj), 0)
        return diag(bi)

    jax.lax.fori_loop(0, _NBLK, outer, 0)

    li = jax.lax.broadcasted_iota(jnp.int32, (_BB, _BB), 0)
    lj = jax.lax.broadcasted_iota(jnp.int32, (_BB, _BB), 1)
    eye = (li == lj).astype(jnp.float32)

    def writeout(b, carry):
        cc = jnp.sum(eye * accc_ref[b], axis=1, keepdims=True)
        sup = jnp.maximum(accr_ref[b], cc)
        out_ref[b] = jnp.where(sup > 0.0, 0.0, rows_ref[b])
        return 0

    jax.lax.fori_loop(0, _NBLK, writeout, 0)


def kernel(boxes, scores):
    data = jnp.zeros((_NPAD, 8), dtype=jnp.float32)
    data = data.at[:_N, 0:4].set(boxes)
    data = data.at[:_N, 4].set(scores)
    rows = data.reshape(_NBLK, _BB, 8)
    cols = jnp.moveaxis(data.T.reshape(8, _NBLK, _BB), 1, 0)

    out = pl.pallas_call(
        _nms_kernel,
        out_shape=jax.ShapeDtypeStruct((_NBLK, _BB, 8), jnp.float32),
        scratch_shapes=[
            pltpu.VMEM((_NBLK, _BB, 1), jnp.float32),
            pltpu.VMEM((_NBLK, 1, _BB), jnp.float32),
        ],
    )(rows, cols)

    return out.reshape(_NPAD, 8)[:_N, :5]


# speedup vs baseline: 1.9247x; 1.1121x over previous
import jax
import jax.numpy as jnp
from jax.experimental import pallas as pl
from jax.experimental.pallas import tpu as pltpu

_N = 5000
_BB = 1280
_NBLK = 4
_NPAD = _BB * _NBLK
_IOU_THRESH = 0.5


def _iou_block(rows, cols):
    rx1 = rows[:, 0:1]
    ry1 = rows[:, 1:2]
    rx2 = rows[:, 2:3]
    ry2 = rows[:, 3:4]
    cx1 = cols[0:1, :]
    cy1 = cols[1:2, :]
    cx2 = cols[2:3, :]
    cy2 = cols[3:4, :]
    r_area = (rx2 - rx1) * (ry2 - ry1)
    c_area = (cx2 - cx1) * (cy2 - cy1)
    w = jnp.maximum(jnp.minimum(rx2, cx2) - jnp.maximum(rx1, cx1), 0.0)
    h = jnp.maximum(jnp.minimum(ry2, cy2) - jnp.maximum(ry1, cy1), 0.0)
    inter = w * h
    union = (r_area + c_area) - inter
    return inter / union


def _nms_kernel(rows_ref, cols_ref, out_ref, accr_ref, accc_ref):
    accr_ref[...] = jnp.zeros_like(accr_ref)
    accc_ref[...] = jnp.zeros_like(accc_ref)

    def offdiag(bi, bj):
        rows = rows_ref[bi]
        cols = cols_ref[bj]
        rs = rows[:, 4:5]
        cs = cols[4:5, :]
        iou = _iou_block(rows, cols)
        col_beats_row = cs >= rs
        row_max = jnp.max(jnp.where(col_beats_row, iou, 0.0), axis=1,
                          keepdims=True)
        col_max = jnp.max(jnp.where(col_beats_row, 0.0, iou), axis=0,
                          keepdims=True)
        accr_ref[bi] = jnp.maximum(accr_ref[bi], row_max)
        accc_ref[bj] = jnp.maximum(accc_ref[bj], col_max)
        return 0

    def diag(bi):
        rows = rows_ref[bi]
        cols = cols_ref[bi]
        rs = rows[:, 4:5]
        cs = cols[4:5, :]
        iou = _iou_block(rows, cols)
        li = jax.lax.broadcasted_iota(jnp.int32, (_BB, 1), 0)
        lj = jax.lax.broadcasted_iota(jnp.int32, (1, _BB), 1)
        col_beats_row = (cs > rs) | ((cs == rs) & (lj < li))
        row_max = jnp.max(jnp.where(col_beats_row, iou, 0.0), axis=1,
                          keepdims=True)
        accr_ref[bi] = jnp.maximum(accr_ref[bi], row_max)
        return 0

    def outer(bi, carry):
        jax.lax.fori_loop(0, bi, lambda bj, c: offdiag(bi, bj), 0)
        return diag(bi)

    jax.lax.fori_loop(0, _NBLK, outer, 0)

    li = jax.lax.broadcasted_iota(jnp.int32, (_BB, _BB), 0)
    lj = jax.lax.broadcasted_iota(jnp.int32, (_BB, _BB), 1)
    eye = (li == lj).astype(jnp.float32)

    def writeout(b, carry):
        cc = jnp.sum(eye * accc_ref[b], axis=1, keepdims=True)
        sup = jnp.maximum(accr_ref[b], cc) > _IOU_THRESH
        out_ref[b] = jnp.where(sup, 0.0, rows_ref[b])
        return 0

    jax.lax.fori_loop(0, _NBLK, writeout, 0)


def kernel(boxes, scores):
    data = jnp.zeros((_NPAD, 8), dtype=jnp.float32)
    data = data.at[:_N, 0:4].set(boxes)
    data = data.at[:_N, 4].set(scores)
    rows = data.reshape(_NBLK, _BB, 8)
    cols = jnp.moveaxis(data.T.reshape(8, _NBLK, _BB), 1, 0)

    out = pl.pallas_call(
        _nms_kernel,
        out_shape=jax.ShapeDtypeStruct((_NBLK, _BB, 8), jnp.float32),
        scratch_shapes=[
            pltpu.VMEM((_NBLK, _BB, 1), jnp.float32),
            pltpu.VMEM((_NBLK, 1, _BB), jnp.float32),
        ],
    )(rows, cols)

    return out.reshape(_NPAD, 8)[:_N, :5]
